# initial kernel scaffold (unmeasured)
import jax
import jax.numpy as jnp
from jax import lax
from jax.experimental import pallas as pl
from jax.experimental.pallas import tpu as pltpu

N_DEV = 4
SCALE = 0.08838834764831843
BLK = 64


def kernel(x, Wq, K_ext, V_ext, Wo):
    B, Sq, D = x.shape
    _, Skv_l, Hq, Dh = K_ext.shape

    x2 = x.reshape(Sq, D).astype(jnp.bfloat16)
    wq = Wq.astype(jnp.bfloat16)
    k2 = K_ext.reshape(Skv_l, Hq, Dh).astype(jnp.bfloat16)
    v2 = V_ext.reshape(Skv_l, Hq, Dh).astype(jnp.bfloat16)
    wo = Wo.astype(jnp.bfloat16)

    def body(x_ref, wq_ref, k_ref, v_ref, wo_ref, out_ref,
             o_comm, s_comm, acc_o, acc_s,
             o_send_sems, o_recv_sems, s_send_sems, s_recv_sems):
        my = lax.axis_index("i")
        left = lax.rem(my + N_DEV - 1, N_DEV)
        right = lax.rem(my + 1, N_DEV)

        barrier_sem = pltpu.get_barrier_semaphore()
        for nbr in (left, right):
            pl.semaphore_signal(barrier_sem, inc=1, device_id=(nbr,),
                                device_id_type=pl.DeviceIdType.MESH)
        pl.semaphore_wait(barrier_sem, 2)

        q = jnp.dot(x_ref[...], wq_ref[...],
                    preferred_element_type=jnp.float32)
        qb = q.astype(jnp.bfloat16)

        q_blk = lax.broadcasted_iota(jnp.int32, (Sq, Skv_l), 0) // BLK
        k_blk = (lax.broadcasted_iota(jnp.int32, (Sq, Skv_l), 1)
                 + my * Skv_l) // BLK
        mask = ((q_blk == k_blk) | (k_blk == 0)
                | (lax.rem(q_blk + k_blk, 3) == 0))

        for h in range(Hq):
            qh = qb[:, h * Dh:(h + 1) * Dh]
            kh = k_ref[:, h, :]
            vh = v_ref[:, h, :]
            s = lax.dot_general(qh, kh, (((1,), (1,)), ((), ())),
                                preferred_element_type=jnp.float32)
            s = jnp.where(mask, s * SCALE, -1e9)
            m = jnp.max(s, axis=1, keepdims=True)
            w = jnp.where(mask, jnp.exp(s - m), 0.0)
            lsum = jnp.sum(w, axis=1, keepdims=True)
            o = jnp.dot(w.astype(jnp.bfloat16), vh,
                        preferred_element_type=jnp.float32)
            acc_o[h] = o
            o_comm[0, h] = o.astype(jnp.bfloat16)
            acc_s[0, pl.ds(h, 1), :] = m.reshape(1, Sq)
            acc_s[1, pl.ds(h, 1), :] = lsum.reshape(1, Sq)
            s_comm[0, 0, pl.ds(h, 1), :] = m.reshape(1, Sq)
            s_comm[0, 1, pl.ds(h, 1), :] = lsum.reshape(1, Sq)

        for hop in range(N_DEV - 1):
            send_slot = hop % 2
            recv_slot = (hop + 1) % 2
            rdma_o = pltpu.make_async_remote_copy(
                src_ref=o_comm.at[send_slot],
                dst_ref=o_comm.at[recv_slot],
                send_sem=o_send_sems.at[send_slot],
                recv_sem=o_recv_sems.at[recv_slot],
                device_id=(right,),
                device_id_type=pl.DeviceIdType.MESH,
            )
            rdma_s = pltpu.make_async_remote_copy(
                src_ref=s_comm.at[send_slot],
                dst_ref=s_comm.at[recv_slot],
                send_sem=s_send_sems.at[send_slot],
                recv_sem=s_recv_sems.at[recv_slot],
                device_id=(right,),
                device_id_type=pl.DeviceIdType.MESH,
            )
            rdma_o.start()
            rdma_s.start()
            rdma_o.wait()
            rdma_s.wait()

            m_r = s_comm[recv_slot, 0]
            l_r = s_comm[recv_slot, 1]
            o_r = o_comm[recv_slot].astype(jnp.float32)
            m_a = acc_s[0]
            l_a = acc_s[1]
            m_new = jnp.maximum(m_a, m_r)
            a = jnp.exp(m_a - m_new)
            b = jnp.exp(m_r - m_new)
            acc_s[0] = m_new
            acc_s[1] = l_a * a + l_r * b
            acc_o[...] = acc_o[...] * a[:, :, None] + o_r * b[:, :, None]

        ctx = (acc_o[...] / acc_s[1][:, :, None]).astype(jnp.bfloat16)
        out = jnp.zeros((Sq, D), jnp.float32)
        for h in range(Hq):
            out = out + jnp.dot(ctx[h], wo_ref[h * Dh:(h + 1) * Dh, :],
                                preferred_element_type=jnp.float32)
        out_ref[...] = out

    out = pl.pallas_call(
        body,
        out_shape=jax.ShapeDtypeStruct((Sq, D), jnp.float32),
        in_specs=[pl.BlockSpec(memory_space=pltpu.VMEM)] * 5,
        out_specs=pl.BlockSpec(memory_space=pltpu.VMEM),
        scratch_shapes=[
            pltpu.VMEM((2, Hq, Sq, Dh), jnp.bfloat16),
            pltpu.VMEM((2, 2, Hq, Sq), jnp.float32),
            pltpu.VMEM((Hq, Sq, Dh), jnp.float32),
            pltpu.VMEM((2, Hq, Sq), jnp.float32),
            pltpu.SemaphoreType.DMA((2,)),
            pltpu.SemaphoreType.DMA((2,)),
            pltpu.SemaphoreType.DMA((2,)),
            pltpu.SemaphoreType.DMA((2,)),
        ],
        compiler_params=pltpu.CompilerParams(collective_id=0),
    )(x2, wq, k2, v2, wo)
    return out.reshape(B, Sq, D)


# baseline (device time: 80938 ns/iter reference)
import jax
import jax.numpy as jnp
from jax import lax
from jax.experimental import pallas as pl
from jax.experimental.pallas import tpu as pltpu

N_DEV = 4
SCALE = 0.08838834764831843
BLK = 64


def kernel(x, Wq, K_ext, V_ext, Wo):
    B, Sq, D = x.shape
    _, Skv_l, Hq, Dh = K_ext.shape

    x2 = x.reshape(Sq, D).astype(jnp.bfloat16)
    wq = Wq.astype(jnp.bfloat16)
    k2 = K_ext.reshape(Skv_l, Hq, Dh).transpose(1, 0, 2).astype(jnp.bfloat16)
    v2 = V_ext.reshape(Skv_l, Hq, Dh).transpose(1, 0, 2).astype(jnp.bfloat16)
    wo = Wo.reshape(Hq, Dh, D).astype(jnp.bfloat16)

    def body(x_ref, wq_ref, k_ref, v_ref, wo_ref, out_ref,
             q_buf, o_comm, s_comm, acc_o, acc_s,
             o_send_sems, o_recv_sems, s_send_sems, s_recv_sems):
        my = lax.axis_index("i")
        left = lax.rem(my + N_DEV - 1, N_DEV)
        right = lax.rem(my + 1, N_DEV)

        barrier_sem = pltpu.get_barrier_semaphore()
        for nbr in (left, right):
            pl.semaphore_signal(barrier_sem, inc=1, device_id=(nbr,),
                                device_id_type=pl.DeviceIdType.MESH)
        pl.semaphore_wait(barrier_sem, 2)

        q = jnp.dot(x_ref[...], wq_ref[...],
                    preferred_element_type=jnp.float32)
        for h in range(Hq):
            q_buf[h] = q[:, h * Dh:(h + 1) * Dh].astype(jnp.bfloat16)

        q_blk = lax.broadcasted_iota(jnp.int32, (Skv_l, Sq), 1) // BLK
        k_blk = (lax.broadcasted_iota(jnp.int32, (Skv_l, Sq), 0)
                 + my * Skv_l) // BLK
        mask = ((q_blk == k_blk) | (k_blk == 0)
                | (lax.rem(q_blk + k_blk, 3) == 0))

        def head_body(h, _):
            qh = q_buf[h]
            s = lax.dot_general(k_ref[h], qh,
                                (((1,), (1,)), ((), ())),
                                preferred_element_type=jnp.float32)
            s = jnp.where(mask, s * SCALE, -1e9)
            m = jnp.max(s, axis=0, keepdims=True)
            w = jnp.where(mask, jnp.exp(s - m), 0.0)
            lsum = jnp.sum(w, axis=0, keepdims=True)
            o = lax.dot_general(v_ref[h], w.astype(jnp.bfloat16),
                                (((0,), (0,)), ((), ())),
                                preferred_element_type=jnp.float32)
            acc_o[h] = o
            o_comm[0, h] = o.astype(jnp.bfloat16)
            acc_s[h, 0:1, :] = m
            acc_s[h, 1:2, :] = lsum
            s_comm[0, h, 0:1, :] = m
            s_comm[0, h, 1:2, :] = lsum
            return 0

        lax.fori_loop(0, Hq, head_body, 0)

        for hop in range(N_DEV - 1):
            send_slot = hop % 2
            recv_slot = (hop + 1) % 2
            rdma_o = pltpu.make_async_remote_copy(
                src_ref=o_comm.at[send_slot],
                dst_ref=o_comm.at[recv_slot],
                send_sem=o_send_sems.at[send_slot],
                recv_sem=o_recv_sems.at[recv_slot],
                device_id=(right,),
                device_id_type=pl.DeviceIdType.MESH,
            )
            rdma_s = pltpu.make_async_remote_copy(
                src_ref=s_comm.at[send_slot],
                dst_ref=s_comm.at[recv_slot],
                send_sem=s_send_sems.at[send_slot],
                recv_sem=s_recv_sems.at[recv_slot],
                device_id=(right,),
                device_id_type=pl.DeviceIdType.MESH,
            )
            rdma_o.start()
            rdma_s.start()
            rdma_o.wait()
            rdma_s.wait()

            def comb_body(h, _):
                m_a = acc_s[h, 0:1, :]
                l_a = acc_s[h, 1:2, :]
                m_r = s_comm[recv_slot, h, 0:1, :]
                l_r = s_comm[recv_slot, h, 1:2, :]
                m_new = jnp.maximum(m_a, m_r)
                a = jnp.exp(m_a - m_new)
                b = jnp.exp(m_r - m_new)
                acc_s[h, 0:1, :] = m_new
                acc_s[h, 1:2, :] = l_a * a + l_r * b
                acc_o[h] = (acc_o[h] * a
                            + o_comm[recv_slot, h].astype(jnp.float32) * b)
                return 0

            lax.fori_loop(0, Hq, comb_body, 0)

        out_ref[...] = jnp.zeros((Sq, D), jnp.float32)

        def proj_body(h, _):
            ctx_h = (acc_o[h] / acc_s[h, 1:2, :]).astype(jnp.bfloat16)
            out_ref[...] += lax.dot_general(
                ctx_h, wo_ref[h], (((0,), (0,)), ((), ())),
                preferred_element_type=jnp.float32)
            return 0

        lax.fori_loop(0, Hq, proj_body, 0)

    out = pl.pallas_call(
        body,
        out_shape=jax.ShapeDtypeStruct((Sq, D), jnp.float32),
        in_specs=[pl.BlockSpec(memory_space=pltpu.VMEM)] * 5,
        out_specs=pl.BlockSpec(memory_space=pltpu.VMEM),
        scratch_shapes=[
            pltpu.VMEM((Hq, Sq, Dh), jnp.bfloat16),
            pltpu.VMEM((2, Hq, Dh, Sq), jnp.bfloat16),
            pltpu.VMEM((2, Hq, 2, Sq), jnp.float32),
            pltpu.VMEM((Hq, Dh, Sq), jnp.float32),
            pltpu.VMEM((Hq, 2, Sq), jnp.float32),
            pltpu.SemaphoreType.DMA((2,)),
            pltpu.SemaphoreType.DMA((2,)),
            pltpu.SemaphoreType.DMA((2,)),
            pltpu.SemaphoreType.DMA((2,)),
        ],
        compiler_params=pltpu.CompilerParams(collective_id=0),
    )(x2, wq, k2, v2, wo)
    return out.reshape(B, Sq, D)
